# Initial kernel scaffold; baseline (speedup 1.0000x reference)
import jax
import jax.numpy as jnp
from jax.experimental import pallas as pl


def kernel(x, W1, b1, W2, b2, Wg, att_src, att_dst, bg):
    raise NotImplementedError("write your pallas kernel here")


# speedup vs baseline: 2.2254x; 2.2254x over previous
import numpy as np
import jax
import jax.numpy as jnp
from jax.experimental import pallas as pl
from jax.experimental.pallas import tpu as pltpu

N = 1024
D = 128
HEADS = 8
HEAD_DIM = 16
K = 10
KP = 16
KM_ITERS = 20
TEMP = 0.5

_EXP_1_OVER_T = np.float32(np.exp(np.float32(1.0 / TEMP)))


def _dotT(a, b):
    return jax.lax.dot_general(a, b, (((1,), (1,)), ((), ())),
                               preferred_element_type=jnp.float32)


def _fused(x_ref, w1_ref, b1_ref, w2_ref, b2_ref, wg_ref, asrc_ref,
           adst_ref, bg_ref, out_ref, loss_ref):
    X = x_ref[...]
    W1 = w1_ref[...]
    W2 = w2_ref[...]

    Hid = jnp.maximum(
        jnp.dot(X, W1, preferred_element_type=jnp.float32) + b1_ref[...], 0.0)
    Z = jnp.dot(Hid, W2, preferred_element_type=jnp.float32) + b2_ref[...]

    sq = jnp.sum(Z * Z, axis=1, keepdims=True)
    nrm = jnp.maximum(jnp.sqrt(sq), 1e-8)
    ZN = Z / nrm
    S = _dotT(ZN, ZN)
    pos = jnp.sum(ZN * ZN, axis=1, keepdims=True)
    den = 2.0 * jnp.sum(jnp.exp(S * (1.0 / TEMP)), axis=1,
                        keepdims=True) - _EXP_1_OVER_T
    nom = jnp.exp(pos * (1.0 / TEMP))
    loss_ref[...] = jnp.reshape(-jnp.mean(jnp.log(nom / den)), (1, 1))

    ones_n1 = jnp.ones((N, 1), jnp.float32)
    kiota = jax.lax.broadcasted_iota(jnp.int32, (N, KP), 1).astype(jnp.float32)

    def assign_of(cent):
        best = jnp.full((N, 1), jnp.inf, jnp.float32)
        bidx = jnp.zeros((N, 1), jnp.float32)
        for k in range(K):
            ck = cent[k:k + 1, :]
            diff = Z - ck
            d2k = jnp.sum(diff * diff, axis=1, keepdims=True)
            take = d2k < best
            best = jnp.where(take, d2k, best)
            bidx = jnp.where(take, jnp.float32(k), bidx)
        return bidx

    def km_body(_, carry):
        cent, _ = carry
        bidx = assign_of(cent)
        onehot = (kiota == bidx).astype(jnp.float32)
        counts = jax.lax.dot_general(onehot, ones_n1, (((0,), (0,)), ((), ())),
                                     preferred_element_type=jnp.float32)
        centn = jax.lax.dot_general(onehot, Z, (((0,), (0,)), ((), ())),
                                    preferred_element_type=jnp.float32)
        return centn / jnp.maximum(counts, 1.0), bidx

    cent0 = Z[0:KP, :]
    _, bidx = jax.lax.fori_loop(0, KM_ITERS, km_body,
                                (cent0, jnp.zeros((N, 1), jnp.float32)))
    onehot = (kiota == bidx).astype(jnp.float32)
    maskf = _dotT(onehot, onehot)
    valid = maskf > 0.5

    Hm = jnp.dot(Z, wg_ref[...], preferred_element_type=jnp.float32)
    a_dst = jnp.dot(Hm, adst_ref[...], preferred_element_type=jnp.float32)
    a_srcT = jax.lax.dot_general(asrc_ref[...], Hm, (((0,), (1,)), ((), ())),
                                 preferred_element_type=jnp.float32)

    bg = bg_ref[...]
    for h in range(HEADS):
        e = a_dst[:, h:h + 1] + a_srcT[h:h + 1, :]
        e = jnp.where(e >= 0.0, e, 0.2 * e)
        e = jnp.where(valid, e, jnp.float32(-1e9))
        m = jnp.max(e, axis=1, keepdims=True)
        p = jnp.exp(e - m)
        attn = p / jnp.sum(p, axis=1, keepdims=True)
        oh = jnp.dot(attn, Hm[:, h * HEAD_DIM:(h + 1) * HEAD_DIM],
                     preferred_element_type=jnp.float32)
        out_ref[:, h * HEAD_DIM:(h + 1) * HEAD_DIM] = (
            oh + bg[0:1, h * HEAD_DIM:(h + 1) * HEAD_DIM])


def kernel(x, W1, b1, W2, b2, Wg, att_src, att_dst, bg):
    bsz, npatch, nv, plen = x.shape
    X = x.reshape(bsz * npatch, nv * plen)
    eyeH = jnp.eye(HEADS, dtype=jnp.float32)
    Asrc = (att_src[:, :, None] * eyeH[:, None, :]).reshape(D, HEADS)
    Adst = (att_dst[:, :, None] * eyeH[:, None, :]).reshape(D, HEADS)

    out, loss = pl.pallas_call(
        _fused,
        out_shape=[
            jax.ShapeDtypeStruct((N, D), jnp.float32),
            jax.ShapeDtypeStruct((1, 1), jnp.float32),
        ],
    )(X, W1, b1.reshape(1, D), W2, b2.reshape(1, D), Wg, Asrc, Adst,
      bg.reshape(1, D))
    return out.reshape(bsz, npatch, nv, plen), loss.reshape(())
